# baseline (speedup 1.0000x reference)
import math

import jax
import jax.numpy as jnp
from jax.experimental import pallas as pl
from jax.experimental.pallas import tpu as pltpu

B, N, E, H = 128, 256, 128, 128
_BIG = 1e30
_ISQ = 1.0 / math.sqrt(E)


def _rne_bf16(x):
    xi = jax.lax.bitcast_convert_type(x, jnp.uint32)
    r = (xi + jnp.uint32(0x7FFF) + ((xi >> jnp.uint32(16)) & jnp.uint32(1)))
    r = r & jnp.uint32(0xFFFF0000)
    return jax.lax.bitcast_convert_type(r, jnp.float32)


def _two_sum(a, b):
    s = a + b
    bb = s - a
    err = (a - (s - bb)) + (b - bb)
    return s, err


def _mxu_rowdot_T(qbT, rowsT):
    p = qbT * rowsT
    p4 = p.reshape(32, 4, -1)
    s1, e1 = _two_sum(p4[:, 0, :], p4[:, 1, :])
    s2, e2 = _two_sum(p4[:, 2, :], p4[:, 3, :])
    s, e3 = _two_sum(s1, s2)
    c4 = s + ((e1 + e2) + e3)
    c4p = c4.reshape(16, 2, -1)
    g = c4p[:, 0, :] + c4p[:, 1, :]
    acc = g[0:1, :]
    for j in range(1, 16):
        acc = acc + g[j:j + 1, :]
    return acc


def _decode_kernel(ne_ref, netbf_ref, whq_ref, wph_ref, bph_ref, wih_ref,
                   whh_ref, bih_ref, bhh_ref,
                   tours_ref, lp_ref, ent_ref,
                   hidden_ref, visited_ref):
    f32 = jnp.float32

    context = jnp.mean(ne_ref[...], axis=1)
    hidden_ref[...] = jnp.tanh(
        jax.lax.dot(context, wph_ref[...], preferred_element_type=f32)
        + bph_ref[...])
    visited_ref[...] = jnp.zeros((B, N), dtype=f32)
    lp_ref[...] = jnp.zeros((B, 8), dtype=f32)
    ent_ref[...] = jnp.zeros((B, 8), dtype=f32)

    iota_n = jax.lax.broadcasted_iota(jnp.int32, (B, N), 1)

    def step(t, _):
        hidden = hidden_ref[...]
        query = jax.lax.dot(hidden, whq_ref[...],
                            preferred_element_type=f32)
        qbf = _rne_bf16(query)
        sraw = jnp.sum(netbf_ref[...] * qbf[:, :, None], axis=1) * _ISQ
        visited = visited_ref[...]
        sm = jnp.where(visited > 0.5, -_BIG, sraw)
        mtil = jnp.max(sm, axis=1, keepdims=True)
        i1 = jnp.min(jnp.where(sm >= mtil, iota_n, N),
                     axis=1, keepdims=True)
        sm2 = jnp.where(iota_n == i1, -_BIG, sm)
        m2til = jnp.max(sm2, axis=1, keepdims=True)
        i2 = jnp.min(jnp.where(sm2 >= m2til, iota_n, N),
                     axis=1, keepdims=True)
        valid2 = m2til > -1e29

        oh1 = (iota_n == i1).astype(f32)
        oh2 = (iota_n == i2).astype(f32)
        ne = ne_ref[...]
        row1 = jnp.sum(ne * oh1[:, :, None], axis=1)
        row2 = jnp.sum(ne * oh2[:, :, None], axis=1)

        qbT = qbf.T
        r1T = _rne_bf16(row1).T
        r2T = _rne_bf16(row2).T
        s1ex = (_mxu_rowdot_T(qbT, r1T) * _ISQ).T
        s2ex = (_mxu_rowdot_T(qbT, r2T) * _ISQ).T

        e = jnp.exp(sm - mtil)
        z = jnp.sum(e, axis=1, keepdims=True)
        probs = e / z
        ent_t = -jnp.sum(probs * jnp.log(probs + 1e-12),
                         axis=1, keepdims=True)

        p1c = jnp.exp(s1ex - mtil) / z
        p2c = jnp.exp(s2ex - mtil) / z
        win2 = ((p2c > p1c) | ((p2c == p1c) & (i2 < i1))) & valid2
        idx = jnp.where(win2, i2, i1)
        p_win = jnp.where(win2, p2c, p1c)
        logp_t = jnp.log(p_win + 1e-12)

        visited_ref[...] = jnp.maximum(visited,
                                       (iota_n == idx).astype(f32))
        chosen = jnp.where(win2, row2, row1)

        gi = jax.lax.dot(chosen, wih_ref[...],
                         preferred_element_type=f32) + bih_ref[...]
        gh = jax.lax.dot(hidden, whh_ref[...],
                         preferred_element_type=f32) + bhh_ref[...]
        i_r, i_z, i_n = gi[:, :H], gi[:, H:2 * H], gi[:, 2 * H:]
        h_r, h_z, h_n = gh[:, :H], gh[:, H:2 * H], gh[:, 2 * H:]
        r = jax.nn.sigmoid(i_r + h_r)
        zg = jax.nn.sigmoid(i_z + h_z)
        ng = jnp.tanh(i_n + r * h_n)
        hidden_ref[...] = (1.0 - zg) * ng + zg * hidden

        tours_ref[...] = jnp.where(iota_n == t, idx, tours_ref[...])
        lp_ref[:, 0:1] += logp_t
        ent_ref[:, 0:1] += ent_t
        return 0

    jax.lax.fori_loop(0, N, step, 0)


def kernel(node_emb, W_hq, W_ph, b_ph, W_ih, W_hh, b_ih, b_hh, greedy=True):
    del greedy
    net_bf = jnp.transpose(
        node_emb.astype(jnp.bfloat16).astype(jnp.float32), (0, 2, 1))
    tours, lp, ent = pl.pallas_call(
        _decode_kernel,
        out_shape=(
            jax.ShapeDtypeStruct((B, N), jnp.int32),
            jax.ShapeDtypeStruct((B, 8), jnp.float32),
            jax.ShapeDtypeStruct((B, 8), jnp.float32),
        ),
        scratch_shapes=[
            pltpu.VMEM((B, H), jnp.float32),
            pltpu.VMEM((B, N), jnp.float32),
        ],
    )(node_emb, net_bf, W_hq.T, W_ph.T, b_ph[None, :],
      W_ih.T, W_hh.T, b_ih[None, :], b_hh[None, :])
    return tours, lp[:, 0], ent[:, 0]


# speedup vs baseline: 1.7032x; 1.7032x over previous
import math

import jax
import jax.numpy as jnp
from jax.experimental import pallas as pl
from jax.experimental.pallas import tpu as pltpu

B, N, E, H = 128, 256, 128, 128
_BIG = 1e30
_ISQ = 1.0 / math.sqrt(E)


def _rne_bf16(x):
    xi = jax.lax.bitcast_convert_type(x, jnp.uint32)
    r = (xi + jnp.uint32(0x7FFF) + ((xi >> jnp.uint32(16)) & jnp.uint32(1)))
    r = r & jnp.uint32(0xFFFF0000)
    return jax.lax.bitcast_convert_type(r, jnp.float32)


def _two_sum(a, b):
    s = a + b
    bb = s - a
    err = (a - (s - bb)) + (b - bb)
    return s, err


def _mxu_rowdot_T(qbT, rowsT):
    p = qbT * rowsT
    p4 = p.reshape(32, 4, -1)
    s1, e1 = _two_sum(p4[:, 0, :], p4[:, 1, :])
    s2, e2 = _two_sum(p4[:, 2, :], p4[:, 3, :])
    s, e3 = _two_sum(s1, s2)
    c4 = s + ((e1 + e2) + e3)
    c4p = c4.reshape(16, 2, -1)
    g = c4p[:, 0, :] + c4p[:, 1, :]
    acc = g[0:1, :]
    for j in range(1, 16):
        acc = acc + g[j:j + 1, :]
    return acc


def _decode_kernel(ne_ref, netbf_ref, whq_ref, wph_ref, bph_ref, wih_ref,
                   whh_ref, bih_ref, bhh_ref,
                   tours_ref, lp_ref, ent_ref,
                   hidden_ref, visited_ref, idx_scr, row_scr):
    f32 = jnp.float32

    context = jnp.mean(ne_ref[...], axis=1)
    hidden_ref[...] = jnp.tanh(
        jax.lax.dot(context, wph_ref[...], preferred_element_type=f32)
        + bph_ref[...])
    visited_ref[...] = jnp.zeros((B, N), dtype=f32)
    lp_ref[...] = jnp.zeros((B, 8), dtype=f32)
    ent_ref[...] = jnp.zeros((B, 8), dtype=f32)

    iota_n = jax.lax.broadcasted_iota(jnp.int32, (B, N), 1)

    def step(t, _):
        hidden = hidden_ref[...]
        query = jax.lax.dot(hidden, whq_ref[...],
                            preferred_element_type=f32)
        qbf = _rne_bf16(query)
        sraw = jnp.sum(netbf_ref[...] * qbf[:, :, None], axis=1) * _ISQ
        visited = visited_ref[...]
        sm = jnp.where(visited > 0.5, -_BIG, sraw)
        mtil = jnp.max(sm, axis=1, keepdims=True)
        i1 = jnp.min(jnp.where(sm >= mtil, iota_n, N),
                     axis=1, keepdims=True)
        sm2 = jnp.where(iota_n == i1, -_BIG, sm)
        m2til = jnp.max(sm2, axis=1, keepdims=True)
        i2 = jnp.min(jnp.where(sm2 >= m2til, iota_n, N),
                     axis=1, keepdims=True)
        valid2 = m2til > -1e29

        idx_scr[:, :, 0:1] = i1[:, None, :]
        idx_scr[:, :, 1:2] = i2[:, None, :]

        def gather_b(b, carry):
            j1 = idx_scr[b, 0, 0]
            j2 = idx_scr[b, 0, 1]
            row_scr[b, :, :E] = ne_ref[b, pl.ds(j1, 1), :]
            row_scr[b, :, E:] = ne_ref[b, pl.ds(j2, 1), :]
            return carry

        jax.lax.fori_loop(0, B, gather_b, 0, unroll=4)
        rows = row_scr[:, 0, :]
        row1 = rows[:, :E]
        row2 = rows[:, E:]

        qbT = qbf.T
        r1T = _rne_bf16(row1).T
        r2T = _rne_bf16(row2).T
        s1ex = (_mxu_rowdot_T(qbT, r1T) * _ISQ).T
        s2ex = (_mxu_rowdot_T(qbT, r2T) * _ISQ).T

        e = jnp.exp(sm - mtil)
        z = jnp.sum(e, axis=1, keepdims=True)
        probs = e / z
        ent_t = -jnp.sum(probs * jnp.log(probs + 1e-12),
                         axis=1, keepdims=True)

        p1c = jnp.exp(s1ex - mtil) / z
        p2c = jnp.exp(s2ex - mtil) / z
        win2 = ((p2c > p1c) | ((p2c == p1c) & (i2 < i1))) & valid2
        idx = jnp.where(win2, i2, i1)
        p_win = jnp.where(win2, p2c, p1c)
        logp_t = jnp.log(p_win + 1e-12)

        visited_ref[...] = jnp.maximum(visited,
                                       (iota_n == idx).astype(f32))
        chosen = jnp.where(win2, row2, row1)

        gi = jax.lax.dot(chosen, wih_ref[...],
                         preferred_element_type=f32) + bih_ref[...]
        gh = jax.lax.dot(hidden, whh_ref[...],
                         preferred_element_type=f32) + bhh_ref[...]
        i_r, i_z, i_n = gi[:, :H], gi[:, H:2 * H], gi[:, 2 * H:]
        h_r, h_z, h_n = gh[:, :H], gh[:, H:2 * H], gh[:, 2 * H:]
        r = jax.nn.sigmoid(i_r + h_r)
        zg = jax.nn.sigmoid(i_z + h_z)
        ng = jnp.tanh(i_n + r * h_n)
        hidden_ref[...] = (1.0 - zg) * ng + zg * hidden

        tours_ref[...] = jnp.where(iota_n == t, idx, tours_ref[...])
        lp_ref[:, 0:1] += logp_t
        ent_ref[:, 0:1] += ent_t
        return 0

    jax.lax.fori_loop(0, N, step, 0)


def kernel(node_emb, W_hq, W_ph, b_ph, W_ih, W_hh, b_ih, b_hh, greedy=True):
    del greedy
    net_bf = jnp.transpose(
        node_emb.astype(jnp.bfloat16).astype(jnp.float32), (0, 2, 1))
    tours, lp, ent = pl.pallas_call(
        _decode_kernel,
        out_shape=(
            jax.ShapeDtypeStruct((B, N), jnp.int32),
            jax.ShapeDtypeStruct((B, 8), jnp.float32),
            jax.ShapeDtypeStruct((B, 8), jnp.float32),
        ),
        scratch_shapes=[
            pltpu.VMEM((B, H), jnp.float32),
            pltpu.VMEM((B, N), jnp.float32),
            pltpu.VMEM((B, 1, 8), jnp.int32),
            pltpu.VMEM((B, 1, 2 * E), jnp.float32),
        ],
    )(node_emb, net_bf, W_hq.T, W_ph.T, b_ph[None, :],
      W_ih.T, W_hh.T, b_ih[None, :], b_hh[None, :])
    return tours, lp[:, 0], ent[:, 0]


# speedup vs baseline: 2.3870x; 1.4015x over previous
import math

import jax
import jax.numpy as jnp
from jax.experimental import pallas as pl
from jax.experimental.pallas import tpu as pltpu

B, N, E, H = 128, 256, 128, 128
_BIG = 1e30
_ISQ = 1.0 / math.sqrt(E)


def _rne_bf16(x):
    xi = jax.lax.bitcast_convert_type(x, jnp.uint32)
    r = (xi + jnp.uint32(0x7FFF) + ((xi >> jnp.uint32(16)) & jnp.uint32(1)))
    r = r & jnp.uint32(0xFFFF0000)
    return jax.lax.bitcast_convert_type(r, jnp.float32)


def _two_sum(a, b):
    s = a + b
    bb = s - a
    err = (a - (s - bb)) + (b - bb)
    return s, err


def _mxu_rowdot_T(qbT, rowsT):
    p = qbT * rowsT
    p4 = p.reshape(32, 4, -1)
    s1, e1 = _two_sum(p4[:, 0, :], p4[:, 1, :])
    s2, e2 = _two_sum(p4[:, 2, :], p4[:, 3, :])
    s, e3 = _two_sum(s1, s2)
    c4 = s + ((e1 + e2) + e3)
    c4p = c4.reshape(16, 2, -1)
    g = c4p[:, 0, :] + c4p[:, 1, :]
    acc = g[0:1, :]
    for j in range(1, 16):
        acc = acc + g[j:j + 1, :]
    return acc


def _decode_kernel(ne_ref, netbf_ref, whq_ref, wph_ref, bph_ref, wih_ref,
                   whh_ref, bih_ref, bhh_ref,
                   tours_ref, lp_ref, ent_ref,
                   hidden_ref, visited_ref, idx_scr, row_scr):
    f32 = jnp.float32

    context = jnp.mean(ne_ref[...], axis=1)
    hidden_ref[...] = jnp.tanh(
        jax.lax.dot(context, wph_ref[...], preferred_element_type=f32)
        + bph_ref[...])
    visited_ref[...] = jnp.zeros((B, N), dtype=f32)
    lp_ref[...] = jnp.zeros((B, 8), dtype=f32)
    ent_ref[...] = jnp.zeros((B, 8), dtype=f32)

    iota_n = jax.lax.broadcasted_iota(jnp.int32, (B, N), 1)

    def step(t, _):
        hidden = hidden_ref[...]
        query = jax.lax.dot(hidden, whq_ref[...],
                            preferred_element_type=f32)
        qbf = _rne_bf16(query)
        netbf = netbf_ref[...].astype(f32)
        sraw = jnp.sum(netbf * qbf[:, :, None], axis=1) * _ISQ
        visited = visited_ref[...]
        sm = jnp.where(visited > 0.5, -_BIG, sraw)
        mtil = jnp.max(sm, axis=1, keepdims=True)
        i1 = jnp.min(jnp.where(sm >= mtil, iota_n, N),
                     axis=1, keepdims=True)
        sm2 = jnp.where(iota_n == i1, -_BIG, sm)
        m2til = jnp.max(sm2, axis=1, keepdims=True)
        i2 = jnp.min(jnp.where(sm2 >= m2til, iota_n, N),
                     axis=1, keepdims=True)
        valid2 = m2til > -1e29

        idx_scr[:, :, 0:1] = (i1 * 512 + i2)[:, None, :]

        def gather_b(b, carry):
            code = idx_scr[b, 0, 0]
            j1 = code // 512
            j2 = code - j1 * 512
            row_scr[b, :, :E] = ne_ref[b, pl.ds(j1, 1), :]
            row_scr[b, :, E:] = ne_ref[b, pl.ds(j2, 1), :]
            return carry

        jax.lax.fori_loop(0, B, gather_b, 0, unroll=8)
        rows = row_scr[:, 0, :]
        row1 = rows[:, :E]
        row2 = rows[:, E:]

        qbT = qbf.T
        r1T = _rne_bf16(row1).T
        r2T = _rne_bf16(row2).T
        s1ex = (_mxu_rowdot_T(qbT, r1T) * _ISQ).T
        s2ex = (_mxu_rowdot_T(qbT, r2T) * _ISQ).T

        e = jnp.exp(sm - mtil)
        z = jnp.sum(e, axis=1, keepdims=True)
        probs = e / z
        ent_t = -jnp.sum(probs * jnp.log(probs + 1e-12),
                         axis=1, keepdims=True)

        p1c = jnp.exp(s1ex - mtil) / z
        p2c = jnp.exp(s2ex - mtil) / z
        win2 = ((p2c > p1c) | ((p2c == p1c) & (i2 < i1))) & valid2
        idx = jnp.where(win2, i2, i1)
        p_win = jnp.where(win2, p2c, p1c)
        logp_t = jnp.log(p_win + 1e-12)

        visited_ref[...] = jnp.maximum(visited,
                                       (iota_n == idx).astype(f32))
        chosen = jnp.where(win2, row2, row1)

        gi = jax.lax.dot(chosen, wih_ref[...],
                         preferred_element_type=f32) + bih_ref[...]
        gh = jax.lax.dot(hidden, whh_ref[...],
                         preferred_element_type=f32) + bhh_ref[...]
        i_r, i_z, i_n = gi[:, :H], gi[:, H:2 * H], gi[:, 2 * H:]
        h_r, h_z, h_n = gh[:, :H], gh[:, H:2 * H], gh[:, 2 * H:]
        r = jax.nn.sigmoid(i_r + h_r)
        zg = jax.nn.sigmoid(i_z + h_z)
        ng = jnp.tanh(i_n + r * h_n)
        hidden_ref[...] = (1.0 - zg) * ng + zg * hidden

        tours_ref[...] = jnp.where(iota_n == t, idx, tours_ref[...])
        lp_ref[:, 0:1] += logp_t
        ent_ref[:, 0:1] += ent_t
        return 0

    jax.lax.fori_loop(0, N, step, 0)


def kernel(node_emb, W_hq, W_ph, b_ph, W_ih, W_hh, b_ih, b_hh, greedy=True):
    del greedy
    net_bf = jnp.transpose(node_emb.astype(jnp.bfloat16), (0, 2, 1))
    tours, lp, ent = pl.pallas_call(
        _decode_kernel,
        out_shape=(
            jax.ShapeDtypeStruct((B, N), jnp.int32),
            jax.ShapeDtypeStruct((B, 8), jnp.float32),
            jax.ShapeDtypeStruct((B, 8), jnp.float32),
        ),
        scratch_shapes=[
            pltpu.VMEM((B, H), jnp.float32),
            pltpu.VMEM((B, N), jnp.float32),
            pltpu.VMEM((B, 1, 8), jnp.int32),
            pltpu.VMEM((B, 1, 2 * E), jnp.float32),
        ],
    )(node_emb, net_bf, W_hq.T, W_ph.T, b_ph[None, :],
      W_ih.T, W_hh.T, b_ih[None, :], b_hh[None, :])
    return tours, lp[:, 0], ent[:, 0]


# speedup vs baseline: 2.5275x; 1.0589x over previous
import math

import jax
import jax.numpy as jnp
from jax.experimental import pallas as pl
from jax.experimental.pallas import tpu as pltpu

B, N, E, H = 128, 256, 128, 128
_BIG = 1e30
_ISQ = 1.0 / math.sqrt(E)


def _rne_bf16(x):
    xi = jax.lax.bitcast_convert_type(x, jnp.uint32)
    r = (xi + jnp.uint32(0x7FFF) + ((xi >> jnp.uint32(16)) & jnp.uint32(1)))
    r = r & jnp.uint32(0xFFFF0000)
    return jax.lax.bitcast_convert_type(r, jnp.float32)


def _two_sum(a, b):
    s = a + b
    bb = s - a
    err = (a - (s - bb)) + (b - bb)
    return s, err


def _mxu_rowdot_T(qbT, rowsT):
    p = qbT * rowsT
    p4 = p.reshape(32, 4, -1)
    s1, e1 = _two_sum(p4[:, 0, :], p4[:, 1, :])
    s2, e2 = _two_sum(p4[:, 2, :], p4[:, 3, :])
    s, e3 = _two_sum(s1, s2)
    c4 = s + ((e1 + e2) + e3)
    c4p = c4.reshape(16, 2, -1)
    g = c4p[:, 0, :] + c4p[:, 1, :]
    acc = g[0:1, :]
    for j in range(1, 16):
        acc = acc + g[j:j + 1, :]
    return acc


def _decode_kernel(ne_ref, netbf_ref, whq_ref, wph_ref, bph_ref, wih_ref,
                   whh_ref, bih_ref, bhh_ref,
                   tours_ref, lp_ref, ent_ref,
                   hidden_ref, visited_ref, idx_scr, row_scr):
    f32 = jnp.float32

    context = jnp.mean(ne_ref[...], axis=1)
    hidden_ref[...] = jnp.tanh(
        jax.lax.dot(context, wph_ref[...], preferred_element_type=f32)
        + bph_ref[...])
    visited_ref[...] = jnp.zeros((B, N), dtype=f32)
    lp_ref[...] = jnp.zeros((B, 8), dtype=f32)
    ent_ref[...] = jnp.zeros((B, 8), dtype=f32)

    iota_n = jax.lax.broadcasted_iota(jnp.int32, (B, N), 1)

    def step(t, _):
        hidden = hidden_ref[...]
        query = jax.lax.dot(hidden, whq_ref[...],
                            preferred_element_type=f32)
        qbf = _rne_bf16(query)
        netbf = netbf_ref[...].astype(f32)
        sraw = jnp.sum(netbf * qbf[:, :, None], axis=1) * _ISQ
        visited = visited_ref[...]
        sm = jnp.where(visited > 0.5, -_BIG, sraw)
        mtil = jnp.max(sm, axis=1, keepdims=True)
        i1 = jnp.min(jnp.where(sm >= mtil, iota_n, N),
                     axis=1, keepdims=True)
        sm2 = jnp.where(iota_n == i1, -_BIG, sm)
        m2til = jnp.max(sm2, axis=1, keepdims=True)
        i2 = jnp.min(jnp.where(sm2 >= m2til, iota_n, N),
                     axis=1, keepdims=True)
        valid2 = m2til > -1e29

        idx_scr[:, :, 0:1] = (i1 * 512 + i2)[:, None, :]

        def gather_b(b, carry):
            code = idx_scr[b, 0, 0]
            j1 = code // 512
            j2 = code - j1 * 512
            row_scr[b, :, :E] = ne_ref[b, pl.ds(j1, 1), :]
            row_scr[b, :, E:] = ne_ref[b, pl.ds(j2, 1), :]
            return carry

        jax.lax.fori_loop(0, B, gather_b, 0, unroll=16)
        rows = row_scr[:, 0, :]
        row1 = rows[:, :E]
        row2 = rows[:, E:]

        qbT = qbf.T
        r1T = _rne_bf16(row1).T
        r2T = _rne_bf16(row2).T
        s1ex = (_mxu_rowdot_T(qbT, r1T) * _ISQ).T
        s2ex = (_mxu_rowdot_T(qbT, r2T) * _ISQ).T

        e = jnp.exp(sm - mtil)
        z = jnp.sum(e, axis=1, keepdims=True)
        probs = e / z
        ent_t = -jnp.sum(probs * jnp.log(probs + 1e-12),
                         axis=1, keepdims=True)

        p1c = jnp.exp(s1ex - mtil) / z
        p2c = jnp.exp(s2ex - mtil) / z
        win2 = ((p2c > p1c) | ((p2c == p1c) & (i2 < i1))) & valid2
        idx = jnp.where(win2, i2, i1)
        p_win = jnp.where(win2, p2c, p1c)
        logp_t = jnp.log(p_win + 1e-12)

        visited_ref[...] = jnp.maximum(visited,
                                       (iota_n == idx).astype(f32))
        chosen = jnp.where(win2, row2, row1)

        gi = jax.lax.dot(chosen, wih_ref[...],
                         preferred_element_type=f32) + bih_ref[...]
        gh = jax.lax.dot(hidden, whh_ref[...],
                         preferred_element_type=f32) + bhh_ref[...]
        i_r, i_z, i_n = gi[:, :H], gi[:, H:2 * H], gi[:, 2 * H:]
        h_r, h_z, h_n = gh[:, :H], gh[:, H:2 * H], gh[:, 2 * H:]
        r = jax.nn.sigmoid(i_r + h_r)
        zg = jax.nn.sigmoid(i_z + h_z)
        ng = jnp.tanh(i_n + r * h_n)
        hidden_ref[...] = (1.0 - zg) * ng + zg * hidden

        tours_ref[...] = jnp.where(iota_n == t, idx, tours_ref[...])
        lp_ref[:, 0:1] += logp_t
        ent_ref[:, 0:1] += ent_t
        return 0

    jax.lax.fori_loop(0, N, step, 0)


def kernel(node_emb, W_hq, W_ph, b_ph, W_ih, W_hh, b_ih, b_hh, greedy=True):
    del greedy
    net_bf = jnp.transpose(node_emb.astype(jnp.bfloat16), (0, 2, 1))
    tours, lp, ent = pl.pallas_call(
        _decode_kernel,
        out_shape=(
            jax.ShapeDtypeStruct((B, N), jnp.int32),
            jax.ShapeDtypeStruct((B, 8), jnp.float32),
            jax.ShapeDtypeStruct((B, 8), jnp.float32),
        ),
        scratch_shapes=[
            pltpu.VMEM((B, H), jnp.float32),
            pltpu.VMEM((B, N), jnp.float32),
            pltpu.VMEM((B, 1, 8), jnp.int32),
            pltpu.VMEM((B, 1, 2 * E), jnp.float32),
        ],
    )(node_emb, net_bf, W_hq.T, W_ph.T, b_ph[None, :],
      W_ih.T, W_hh.T, b_ih[None, :], b_hh[None, :])
    return tours, lp[:, 0], ent[:, 0]


# speedup vs baseline: 2.6000x; 1.0287x over previous
import math

import jax
import jax.numpy as jnp
from jax.experimental import pallas as pl
from jax.experimental.pallas import tpu as pltpu

B, N, E, H = 128, 256, 128, 128
_BIG = 1e30
_ISQ = 1.0 / math.sqrt(E)


def _rne_bf16(x):
    xi = jax.lax.bitcast_convert_type(x, jnp.uint32)
    r = (xi + jnp.uint32(0x7FFF) + ((xi >> jnp.uint32(16)) & jnp.uint32(1)))
    r = r & jnp.uint32(0xFFFF0000)
    return jax.lax.bitcast_convert_type(r, jnp.float32)


def _opaque(x):
    return jax.lax.bitcast_convert_type(
        jax.lax.bitcast_convert_type(x, jnp.uint32), jnp.float32)


def _two_sum(a, b):
    s = _opaque(a + b)
    bb = _opaque(s - a)
    err = _opaque(a - _opaque(s - bb)) + _opaque(b - bb)
    return s, err


def _mxu_rowdot_T(qbT, rowsT):
    p = qbT * rowsT
    p4 = p.reshape(32, 4, -1)
    s1, e1 = _two_sum(p4[:, 0, :], p4[:, 1, :])
    s2, e2 = _two_sum(p4[:, 2, :], p4[:, 3, :])
    s, e3 = _two_sum(s1, s2)
    c4 = _opaque(s + _opaque(_opaque(e1 + e2) + e3))
    c4p = c4.reshape(16, 2, -1)
    g = _opaque(c4p[:, 0, :] + c4p[:, 1, :])
    acc = g[0:1, :]
    for j in range(1, 16):
        acc = _opaque(acc + g[j:j + 1, :])
    return acc


def _decode_kernel(ne_ref, netbf_ref, whq_ref, wph_ref, bph_ref, wih_ref,
                   whh_ref, bih_ref, bhh_ref,
                   tours_ref, lp_ref, ent_ref,
                   hidden_ref, visited_ref, idx_scr, row_scr, qbf_scr,
                   rbf_scr):
    f32 = jnp.float32

    context = jnp.mean(ne_ref[...], axis=1)
    hidden_ref[...] = jnp.tanh(
        jax.lax.dot(context, wph_ref[...], preferred_element_type=f32)
        + bph_ref[...])
    visited_ref[...] = jnp.zeros((B, N), dtype=f32)
    lp_ref[...] = jnp.zeros((B, 8), dtype=f32)
    ent_ref[...] = jnp.zeros((B, 8), dtype=f32)

    iota_n = jax.lax.broadcasted_iota(jnp.int32, (B, N), 1)

    def step(t, _):
        hidden = hidden_ref[...]
        query = jax.lax.dot(hidden, whq_ref[...],
                            preferred_element_type=f32)
        qbf_scr[...] = query.astype(jnp.bfloat16)
        qbf = qbf_scr[...].astype(f32)
        sraw = jnp.sum(netbf_ref[...] * qbf[:, :, None], axis=1) * _ISQ
        visited = visited_ref[...]
        sm = jnp.where(visited > 0.5, -_BIG, sraw)
        mtil = jnp.max(sm, axis=1, keepdims=True)
        i1 = jnp.min(jnp.where(sm >= mtil, iota_n, N),
                     axis=1, keepdims=True)
        sm2 = jnp.where(iota_n == i1, -_BIG, sm)
        m2til = jnp.max(sm2, axis=1, keepdims=True)
        i2 = jnp.min(jnp.where(sm2 >= m2til, iota_n, N),
                     axis=1, keepdims=True)
        valid2 = m2til > -1e29

        idx_scr[:, :, 0:1] = (i1 * 512 + i2)[:, None, :]

        def gather_b(b, carry):
            code = idx_scr[b, 0, 0]
            j1 = code // 512
            j2 = code - j1 * 512
            row_scr[b, :, :E] = ne_ref[b, pl.ds(j1, 1), :]
            row_scr[b, :, E:] = ne_ref[b, pl.ds(j2, 1), :]
            return carry

        jax.lax.fori_loop(0, B, gather_b, 0, unroll=16)
        rows = row_scr[:, 0, :]
        row1 = rows[:, :E]
        row2 = rows[:, E:]

        qbT = qbf.T
        rbf_scr[...] = rows.astype(jnp.bfloat16)
        rows_bf = rbf_scr[...].astype(f32)
        r1T = rows_bf[:, :E].T
        r2T = rows_bf[:, E:].T
        s1ex = (_mxu_rowdot_T(qbT, r1T) * _ISQ).T
        s2ex = (_mxu_rowdot_T(qbT, r2T) * _ISQ).T

        e = jnp.exp(sm - mtil)
        z = jnp.sum(e, axis=1, keepdims=True)
        probs = e / z
        ent_t = -jnp.sum(probs * jnp.log(probs + 1e-12),
                         axis=1, keepdims=True)

        m_ex = jnp.maximum(s1ex, s2ex)
        e1x = jnp.exp(s1ex - m_ex)
        e2x = jnp.exp(s2ex - m_ex)
        zx = z * jnp.exp(mtil - m_ex)
        d1 = e1x / zx
        d2 = e2x / zx
        win2 = ((d2 > d1) | ((d2 == d1) & (i2 < i1))) & valid2
        idx = jnp.where(win2, i2, i1)
        p_win = jnp.exp(jnp.where(win2, s2ex, s1ex) - mtil) / z
        logp_t = jnp.log(p_win + 1e-12)

        visited_ref[...] = jnp.maximum(visited,
                                       (iota_n == idx).astype(f32))
        chosen = jnp.where(win2, row2, row1)

        gi = jax.lax.dot(chosen, wih_ref[...],
                         preferred_element_type=f32) + bih_ref[...]
        gh = jax.lax.dot(hidden, whh_ref[...],
                         preferred_element_type=f32) + bhh_ref[...]
        i_r, i_z, i_n = gi[:, :H], gi[:, H:2 * H], gi[:, 2 * H:]
        h_r, h_z, h_n = gh[:, :H], gh[:, H:2 * H], gh[:, 2 * H:]
        r = jax.nn.sigmoid(i_r + h_r)
        zg = jax.nn.sigmoid(i_z + h_z)
        ng = jnp.tanh(i_n + r * h_n)
        hidden_ref[...] = (1.0 - zg) * ng + zg * hidden

        tours_ref[...] = jnp.where(iota_n == t, idx, tours_ref[...])
        lp_ref[:, 0:1] += logp_t
        ent_ref[:, 0:1] += ent_t
        return 0

    jax.lax.fori_loop(0, N, step, 0)


def kernel(node_emb, W_hq, W_ph, b_ph, W_ih, W_hh, b_ih, b_hh, greedy=True):
    del greedy
    net_bf = jnp.transpose(
        node_emb.astype(jnp.bfloat16).astype(jnp.float32), (0, 2, 1))
    tours, lp, ent = pl.pallas_call(
        _decode_kernel,
        out_shape=(
            jax.ShapeDtypeStruct((B, N), jnp.int32),
            jax.ShapeDtypeStruct((B, 8), jnp.float32),
            jax.ShapeDtypeStruct((B, 8), jnp.float32),
        ),
        scratch_shapes=[
            pltpu.VMEM((B, H), jnp.float32),
            pltpu.VMEM((B, N), jnp.float32),
            pltpu.VMEM((B, 1, 8), jnp.int32),
            pltpu.VMEM((B, 1, 2 * E), jnp.float32),
            pltpu.VMEM((B, E), jnp.bfloat16),
            pltpu.VMEM((B, 2 * E), jnp.bfloat16),
        ],
    )(node_emb, net_bf, W_hq.T, W_ph.T, b_ph[None, :],
      W_ih.T, W_hh.T, b_ih[None, :], b_hh[None, :])
    return tours, lp[:, 0], ent[:, 0]
